# baseline (speedup 1.0000x reference)
import functools

import jax
import jax.numpy as jnp
from jax.experimental import pallas as pl
from jax.experimental.pallas import tpu as pltpu

VOCAB = 8192
DIM = 8192
SUB = 8
LANE = DIM // SUB
NLOOKUP = 8192
ROWS_PER_STEP = 4


def _tc_gather(flat_idx, table3):
    grid = (NLOOKUP // ROWS_PER_STEP,)

    in_specs = [
        pl.BlockSpec(
            (1, SUB, LANE),
            functools.partial(
                lambda k, i, idx_ref: (idx_ref[i * ROWS_PER_STEP + k], 0, 0), k
            ),
        )
        for k in range(ROWS_PER_STEP)
    ]

    def out_map(i, idx_ref):
        del idx_ref
        return (i, 0, 0)

    def body(idx_ref, *refs):
        row_refs = refs[:ROWS_PER_STEP]
        out_ref = refs[ROWS_PER_STEP]
        for k in range(ROWS_PER_STEP):
            out_ref[k, :, :] = row_refs[k][0, :, :]

    return pl.pallas_call(
        body,
        grid_spec=pltpu.PrefetchScalarGridSpec(
            num_scalar_prefetch=1,
            grid=grid,
            in_specs=in_specs,
            out_specs=pl.BlockSpec((ROWS_PER_STEP, SUB, LANE), out_map),
        ),
        out_shape=jax.ShapeDtypeStruct((NLOOKUP, SUB, LANE), jnp.float32),
    )(flat_idx, *([table3] * ROWS_PER_STEP))


def kernel(idx, table):
    flat_idx = idx.reshape(-1).astype(jnp.int32)
    table3 = table.reshape(VOCAB, SUB, LANE)
    out = _tc_gather(flat_idx, table3)
    return out.reshape(idx.shape[0], idx.shape[1], DIM)


# speedup vs baseline: 7.8832x; 7.8832x over previous
import functools

import jax
import jax.numpy as jnp
from jax import lax
from jax.experimental import pallas as pl
from jax.experimental.pallas import tpu as pltpu
from jax.experimental.pallas import tpu_sc as plsc

VOCAB = 8192
DIM = 8192
NLOOKUP = 8192
NWORKER = 32
BPW = NLOOKUP // NWORKER
CHUNK = 4
NSTEP = BPW // CHUNK
NBUF = 3

_mesh = plsc.VectorSubcoreMesh(core_axis_name="c", subcore_axis_name="s")


@functools.partial(
    pl.kernel,
    mesh=_mesh,
    out_type=jax.ShapeDtypeStruct((NLOOKUP, DIM), jnp.float32),
    scratch_types=[
        pltpu.VMEM((NSTEP, CHUNK), jnp.int32),
        pltpu.VMEM((CHUNK, DIM), jnp.float32),
        pltpu.VMEM((CHUNK, DIM), jnp.float32),
        pltpu.VMEM((CHUNK, DIM), jnp.float32),
        pltpu.SemaphoreType.DMA,
        pltpu.SemaphoreType.DMA,
        pltpu.SemaphoreType.DMA,
        pltpu.SemaphoreType.DMA,
        pltpu.SemaphoreType.DMA,
        pltpu.SemaphoreType.DMA,
    ],
)
def _gather(idx_hbm, table_hbm, out_hbm, idx_v, buf0, buf1, buf2,
            g0, g1, g2, o0, o1, o2):
    wid = lax.axis_index("s") * 2 + lax.axis_index("c")
    base = wid * BPW
    pltpu.sync_copy(idx_hbm.at[pl.ds(wid * NSTEP, NSTEP)], idx_v)

    bufs = (buf0, buf1, buf2)
    gsems = (g0, g1, g2)
    osems = (o0, o1, o2)

    def start_gather(s, b):
        pltpu.async_copy(table_hbm.at[idx_v.at[s]], bufs[b], gsems[b])

    def start_out(s, b):
        pltpu.async_copy(
            bufs[b], out_hbm.at[pl.ds(base + s * CHUNK, CHUNK)], osems[b]
        )

    def wait(sem):
        pltpu.make_async_copy(
            out_hbm.at[pl.ds(0, CHUNK)], bufs[0], sem
        ).wait()

    start_gather(0, 0)
    start_gather(1, 1)
    wait(gsems[0])
    start_out(0, 0)
    start_gather(2, 2)
    wait(gsems[1])
    start_out(1, 1)
    wait(osems[0])
    start_gather(3, 0)
    wait(gsems[2])
    start_out(2, 2)
    wait(osems[1])
    start_gather(4, 1)

    def body(k, carry):
        for b in range(NBUF):
            s = 3 * k + b
            wait(gsems[b])
            start_out(s, b)
            wait(osems[(b + 2) % NBUF])
            start_gather(s + 2, (b + 2) % NBUF)
        return carry

    lax.fori_loop(1, 20, body, 0)

    wait(gsems[0])
    start_out(60, 0)
    wait(osems[2])
    start_gather(62, 2)
    wait(gsems[1])
    start_out(61, 1)
    wait(osems[0])
    start_gather(63, 0)
    wait(gsems[2])
    start_out(62, 2)
    wait(gsems[0])
    start_out(63, 0)
    wait(osems[1])
    wait(osems[2])
    wait(osems[0])


def kernel(idx, table):
    flat_idx = idx.reshape(-1).astype(jnp.int32)
    idx2 = flat_idx.reshape(NLOOKUP // CHUNK, CHUNK)
    out = _gather(idx2, table)
    return out.reshape(idx.shape[0], idx.shape[1], DIM)
